# Initial kernel scaffold; baseline (speedup 1.0000x reference)
import jax
import jax.numpy as jnp
from jax.experimental import pallas as pl


def kernel(student_ids, question_ids, theta, alpha, beta):
    raise NotImplementedError("write your pallas kernel here")


# speedup vs baseline: 1.1402x; 1.1402x over previous
import functools

import jax
import jax.numpy as jnp
from jax import lax
from jax.experimental import pallas as pl
from jax.experimental.pallas import tpu as pltpu
from jax.experimental.pallas import tpu_sc as plsc

BATCH_ = 16384
LANES = 16
ROW = 128


def _sc_grid():
    try:
        info = plsc.get_sparse_core_info()
        return info.num_cores, info.num_subcores
    except Exception:
        return 2, 16


def _irt_body(nc, rows_per_w,
              sid_hbm, qid_hbm, theta_hbm, alpha_hbm, beta_hbm, out_hbm,
              idx_s, idx_q, t_v, a_v, b_v, o_v, sem):
    wid = lax.axis_index("s") * nc + lax.axis_index("c")
    r0 = wid * rows_per_w

    pltpu.sync_copy(sid_hbm.at[pl.ds(r0, rows_per_w)], idx_s)
    pltpu.sync_copy(qid_hbm.at[pl.ds(r0, rows_per_w)], idx_q)

    copies = []
    for j in range(rows_per_w):
        copies.append(pltpu.async_copy(theta_hbm.at[idx_s.at[j]], t_v.at[j], sem))
        copies.append(pltpu.async_copy(alpha_hbm.at[idx_q.at[j]], a_v.at[j], sem))
        copies.append(pltpu.async_copy(beta_hbm.at[idx_q.at[j]], b_v.at[j], sem))
    for c in copies:
        c.wait()

    for j in range(rows_per_w):
        for i in range(ROW // LANES):
            sl = pl.ds(i * LANES, LANES)
            t = t_v[j, sl]
            a = a_v[j, sl]
            b = b_v[j, sl]
            x = a * (t - b)
            o_v[j, sl] = 1.0 / (1.0 + jnp.exp(-x))

    pltpu.sync_copy(o_v, out_hbm.at[pl.ds(r0, rows_per_w)])


@jax.jit
def kernel(student_ids, question_ids, theta, alpha, beta):
    nc, ns = _sc_grid()
    nw = nc * ns
    batch = student_ids.shape[0]
    n_rows = batch // ROW
    rows_per_w = n_rows // nw

    sid = student_ids.astype(jnp.int32).reshape(n_rows, ROW)
    qid = question_ids.astype(jnp.int32).reshape(n_rows, ROW)
    theta1 = theta.reshape(-1)
    alpha1 = alpha.reshape(-1)
    beta1 = beta.reshape(-1)

    mesh = plsc.VectorSubcoreMesh(core_axis_name="c", subcore_axis_name="s",
                                  num_cores=nc, num_subcores=ns)
    f = pl.kernel(
        functools.partial(_irt_body, nc, rows_per_w),
        out_type=jax.ShapeDtypeStruct((n_rows, ROW), jnp.float32),
        mesh=mesh,
        scratch_types=[
            pltpu.VMEM((rows_per_w, ROW), jnp.int32),
            pltpu.VMEM((rows_per_w, ROW), jnp.int32),
            pltpu.VMEM((rows_per_w, ROW), jnp.float32),
            pltpu.VMEM((rows_per_w, ROW), jnp.float32),
            pltpu.VMEM((rows_per_w, ROW), jnp.float32),
            pltpu.VMEM((rows_per_w, ROW), jnp.float32),
            pltpu.SemaphoreType.DMA,
        ],
    )
    out = f(sid, qid, theta1, alpha1, beta1)
    return out.reshape(batch, 1)


# speedup vs baseline: 1.1439x; 1.0032x over previous
import functools

import jax
import jax.numpy as jnp
from jax import lax
from jax.experimental import pallas as pl
from jax.experimental.pallas import tpu as pltpu
from jax.experimental.pallas import tpu_sc as plsc

LANES = 16
CHUNK = 128


def _sc_grid():
    try:
        info = plsc.get_sparse_core_info()
        return info.num_cores, info.num_subcores
    except Exception:
        return 2, 16


def _irt_body(nc, b_per_w,
              sid_hbm, qid_hbm, theta_hbm, alpha_hbm, beta_hbm, out_hbm,
              idx_s, idx_q, t_v, a_v, b_v, o_v, sem):
    wid = lax.axis_index("s") * nc + lax.axis_index("c")
    base = wid * b_per_w

    pltpu.sync_copy(sid_hbm.at[pl.ds(base, b_per_w)], idx_s)
    pltpu.sync_copy(qid_hbm.at[pl.ds(base, b_per_w)], idx_q)

    copies = []
    for j in range(b_per_w // CHUNK):
        sl = pl.ds(j * CHUNK, CHUNK)
        copies.append(pltpu.async_copy(theta_hbm.at[idx_s.at[sl]], t_v.at[sl], sem))
        copies.append(pltpu.async_copy(alpha_hbm.at[idx_q.at[sl]], a_v.at[sl], sem))
        copies.append(pltpu.async_copy(beta_hbm.at[idx_q.at[sl]], b_v.at[sl], sem))
    for c in copies:
        c.wait()

    for i in range(b_per_w // LANES):
        sl = pl.ds(i * LANES, LANES)
        t = t_v[sl]
        a = a_v[sl]
        b = b_v[sl]
        x = a * (t - b)
        o_v[sl] = 1.0 / (1.0 + jnp.exp(-x))

    pltpu.sync_copy(o_v, out_hbm.at[pl.ds(base, b_per_w)])


@jax.jit
def kernel(student_ids, question_ids, theta, alpha, beta):
    nc, ns = _sc_grid()
    nw = nc * ns
    batch = student_ids.shape[0]
    b_per_w = batch // nw

    sid = student_ids.astype(jnp.int32)
    qid = question_ids.astype(jnp.int32)
    theta1 = theta.reshape(-1)
    alpha1 = alpha.reshape(-1)
    beta1 = beta.reshape(-1)

    mesh = plsc.VectorSubcoreMesh(core_axis_name="c", subcore_axis_name="s",
                                  num_cores=nc, num_subcores=ns)
    f = pl.kernel(
        functools.partial(_irt_body, nc, b_per_w),
        out_type=jax.ShapeDtypeStruct((batch,), jnp.float32),
        mesh=mesh,
        scratch_types=[
            pltpu.VMEM((b_per_w,), jnp.int32),
            pltpu.VMEM((b_per_w,), jnp.int32),
            pltpu.VMEM((b_per_w,), jnp.float32),
            pltpu.VMEM((b_per_w,), jnp.float32),
            pltpu.VMEM((b_per_w,), jnp.float32),
            pltpu.VMEM((b_per_w,), jnp.float32),
            pltpu.SemaphoreType.DMA,
        ],
    )
    out = f(sid, qid, theta1, alpha1, beta1)
    return out.reshape(batch, 1)


# speedup vs baseline: 3.3862x; 2.9603x over previous
import functools

import jax
import jax.numpy as jnp
from jax import lax
from jax.experimental import pallas as pl
from jax.experimental.pallas import tpu as pltpu
from jax.experimental.pallas import tpu_sc as plsc

LANES = 16
CHUNK = 128


def _sc_grid():
    try:
        info = plsc.get_sparse_core_info()
        return info.num_cores, info.num_subcores
    except Exception:
        return 2, 16


def _irt_body(nc, b_per_w,
              sid_hbm, qid_hbm, theta_hbm, alpha_hbm, beta_hbm, out_hbm,
              idx_s, idx_q, t_v, a_v, b_v, o_v, sem):
    wid = lax.axis_index("s") * nc + lax.axis_index("c")
    base = wid * b_per_w

    pltpu.sync_copy(sid_hbm.at[pl.ds(base, b_per_w)], idx_s)
    pltpu.sync_copy(qid_hbm.at[pl.ds(base, b_per_w)], idx_q)

    theta_f = theta_hbm.at[0, :]
    alpha_f = alpha_hbm.at[0, :]
    beta_f = beta_hbm.at[0, :]

    copies = []
    for j in range(b_per_w // CHUNK):
        sl = pl.ds(j * CHUNK, CHUNK)
        copies.append(pltpu.async_copy(theta_f.at[idx_s.at[sl]], t_v.at[sl], sem))
        copies.append(pltpu.async_copy(alpha_f.at[idx_q.at[sl]], a_v.at[sl], sem))
        copies.append(pltpu.async_copy(beta_f.at[idx_q.at[sl]], b_v.at[sl], sem))
    for c in copies:
        c.wait()

    for i in range(b_per_w // LANES):
        sl = pl.ds(i * LANES, LANES)
        t = t_v[sl]
        a = a_v[sl]
        b = b_v[sl]
        x = a * (t - b)
        o_v[sl] = 1.0 / (1.0 + jnp.exp(-x))

    pltpu.sync_copy(o_v, out_hbm.at[pl.ds(base, b_per_w)])


@jax.jit
def kernel(student_ids, question_ids, theta, alpha, beta):
    nc, ns = _sc_grid()
    nw = nc * ns
    batch = student_ids.shape[0]
    b_per_w = batch // nw

    sid = student_ids.astype(jnp.int32)
    qid = question_ids.astype(jnp.int32)
    theta1 = theta.reshape(1, -1)
    alpha1 = alpha.reshape(1, -1)
    beta1 = beta.reshape(1, -1)

    mesh = plsc.VectorSubcoreMesh(core_axis_name="c", subcore_axis_name="s",
                                  num_cores=nc, num_subcores=ns)
    f = pl.kernel(
        functools.partial(_irt_body, nc, b_per_w),
        out_type=jax.ShapeDtypeStruct((batch,), jnp.float32),
        mesh=mesh,
        scratch_types=[
            pltpu.VMEM((b_per_w,), jnp.int32),
            pltpu.VMEM((b_per_w,), jnp.int32),
            pltpu.VMEM((b_per_w,), jnp.float32),
            pltpu.VMEM((b_per_w,), jnp.float32),
            pltpu.VMEM((b_per_w,), jnp.float32),
            pltpu.VMEM((b_per_w,), jnp.float32),
            pltpu.SemaphoreType.DMA,
        ],
    )
    out = f(sid, qid, theta1, alpha1, beta1)
    return out.reshape(batch, 1)


# speedup vs baseline: 3.4324x; 1.0137x over previous
import functools

import jax
import jax.numpy as jnp
from jax import lax
from jax.experimental import pallas as pl
from jax.experimental.pallas import tpu as pltpu
from jax.experimental.pallas import tpu_sc as plsc

LANES = 16
CHUNK = 128


def _sc_grid():
    try:
        info = plsc.get_sparse_core_info()
        return info.num_cores, info.num_subcores
    except Exception:
        return 2, 16


def _irt_body(nc, b_per_w,
              sid_hbm, qid_hbm, theta_hbm, alpha_hbm, beta_hbm, out_hbm,
              idx_s, idx_q, t_v, a_v, b_v, o_v, sem, semi, semo):
    wid = lax.axis_index("s") * nc + lax.axis_index("c")
    base = wid * b_per_w

    ci = pltpu.async_copy(sid_hbm.at[pl.ds(base, b_per_w)], idx_s, semi)
    cq = pltpu.async_copy(qid_hbm.at[pl.ds(base, b_per_w)], idx_q, semi)

    theta_f = theta_hbm.at[0, :]
    alpha_f = alpha_hbm.at[0, :]
    beta_f = beta_hbm.at[0, :]
    ci.wait()
    cq.wait()

    copies = []
    for j in range(b_per_w // CHUNK):
        sl = pl.ds(j * CHUNK, CHUNK)
        copies.append(pltpu.async_copy(theta_f.at[idx_s.at[sl]], t_v.at[sl], sem))
        copies.append(pltpu.async_copy(alpha_f.at[idx_q.at[sl]], a_v.at[sl], sem))
        copies.append(pltpu.async_copy(beta_f.at[idx_q.at[sl]], b_v.at[sl], sem))
    outs = []
    for j in range(b_per_w // CHUNK):
        for k in range(3):
            copies[j * 3 + k].wait()
        for i in range(CHUNK // LANES):
            sl = pl.ds(j * CHUNK + i * LANES, LANES)
            x = a_v[sl] * (t_v[sl] - b_v[sl])
            o_v[sl] = 1.0 / (1.0 + jnp.exp(-x))
        sl = pl.ds(j * CHUNK, CHUNK)
        outs.append(pltpu.async_copy(
            o_v.at[sl], out_hbm.at[pl.ds(base + j * CHUNK, CHUNK)], semo))
    for c in outs:
        c.wait()


@jax.jit
def kernel(student_ids, question_ids, theta, alpha, beta):
    nc, ns = _sc_grid()
    nw = nc * ns
    batch = student_ids.shape[0]
    b_per_w = batch // nw

    sid = student_ids.astype(jnp.int32)
    qid = question_ids.astype(jnp.int32)
    theta1 = theta.reshape(1, -1)
    alpha1 = alpha.reshape(1, -1)
    beta1 = beta.reshape(1, -1)

    mesh = plsc.VectorSubcoreMesh(core_axis_name="c", subcore_axis_name="s",
                                  num_cores=nc, num_subcores=ns)
    f = pl.kernel(
        functools.partial(_irt_body, nc, b_per_w),
        out_type=jax.ShapeDtypeStruct((batch,), jnp.float32),
        mesh=mesh,
        scratch_types=[
            pltpu.VMEM((b_per_w,), jnp.int32),
            pltpu.VMEM((b_per_w,), jnp.int32),
            pltpu.VMEM((b_per_w,), jnp.float32),
            pltpu.VMEM((b_per_w,), jnp.float32),
            pltpu.VMEM((b_per_w,), jnp.float32),
            pltpu.VMEM((b_per_w,), jnp.float32),
            pltpu.SemaphoreType.DMA,
            pltpu.SemaphoreType.DMA,
            pltpu.SemaphoreType.DMA,
        ],
    )
    out = f(sid, qid, theta1, alpha1, beta1)
    return out.reshape(batch, 1)
